# baseline (speedup 1.0000x reference)
import functools

import jax
import jax.numpy as jnp
from jax import lax
from jax.experimental import pallas as pl
from jax.experimental.pallas import tpu as pltpu
from jax.experimental.pallas import tpu_sc as plsc

_TOPK = 8
_IMIN = -2147483648

_NSUB = 16
_LANES = 16
_E = 64


def _sc_hist_kernel(idx_hbm, w_hbm, out_hbm, idxv, wv, accv, redv, bigv,
                    outv, shared):
    sid = lax.axis_index("s")
    flat = idx_hbm.shape[0]
    chunk = flat // _NSUB
    base = sid * chunk
    pltpu.sync_copy(idx_hbm.at[pl.ds(base, chunk)], idxv)
    pltpu.sync_copy(w_hbm.at[pl.ds(base, chunk)], wv)

    zero = jnp.zeros((_LANES,), jnp.float32)
    for r in range(_LANES):
        for j in range(_E // _LANES):
            accv[r, pl.ds(j * _LANES, _LANES)] = zero

    lane = lax.iota(jnp.int32, _LANES)

    @pl.loop(0, chunk // _LANES, unroll=4)
    def _scatter(i):
        iv = idxv[pl.ds(i * _LANES, _LANES)]
        vv = wv[pl.ds(i * _LANES, _LANES)]
        cur = plsc.load_gather(accv, [lane, iv])
        plsc.store_scatter(accv, [lane, iv], cur + vv)

    for j in range(_E // _LANES):
        tot = accv[0, pl.ds(j * _LANES, _LANES)]
        for r in range(1, _LANES):
            tot = tot + accv[r, pl.ds(j * _LANES, _LANES)]
        redv[pl.ds(j * _LANES, _LANES)] = tot

    pltpu.sync_copy(redv, shared.at[pl.ds(sid * _E, _E)])
    plsc.subcore_barrier()

    @pl.when(sid == 0)
    def _final():
        pltpu.sync_copy(shared, bigv)
        cs = []
        for j in range(_E // _LANES):
            tot = bigv[pl.ds(j * _LANES, _LANES)]
            for r in range(1, _NSUB):
                tot = tot + bigv[pl.ds(r * _E + j * _LANES, _LANES)]
            cs.append(tot)
        mx16 = jnp.maximum(jnp.maximum(cs[0], cs[1]), jnp.maximum(cs[2], cs[3]))
        sm16 = cs[0] + cs[1] + cs[2] + cs[3]
        mx = lax.reduce_max(mx16, axes=(0,))
        avg = lax.reduce_sum(sm16, axes=(0,)) * (1.0 / _E)
        num = jnp.full((_LANES,), mx - avg, jnp.float32)
        den = jnp.full((_LANES,), avg + 1e-5, jnp.float32)
        outv[...] = num / den
        pltpu.sync_copy(outv, out_hbm)


def _sc_maxvio(idx_flat, w_flat):
    mesh = plsc.VectorSubcoreMesh(core_axis_name="c", subcore_axis_name="s",
                                  num_cores=1, num_subcores=_NSUB)
    f = functools.partial(
        pl.kernel,
        mesh=mesh,
        out_type=jax.ShapeDtypeStruct((_LANES,), jnp.float32),
        scratch_types=[
            pltpu.VMEM((idx_flat.shape[0] // _NSUB,), jnp.int32),
            pltpu.VMEM((idx_flat.shape[0] // _NSUB,), jnp.float32),
            pltpu.VMEM((_LANES, _E), jnp.float32),
            pltpu.VMEM((_E,), jnp.float32),
            pltpu.VMEM((_NSUB * _E,), jnp.float32),
            pltpu.VMEM((_LANES,), jnp.float32),
            pltpu.VMEM_SHARED((_NSUB * _E,), jnp.float32),
        ],
        compiler_params=pltpu.CompilerParams(needs_layout_passes=False),
    )(_sc_hist_kernel)
    return f(idx_flat, w_flat)


def _gate_kernel(hs_ref, wt_ref, bias_ref, mask_ref,
                 idx_ref, probs_ref, vio_ref, counts_ref):
    i = pl.program_id(0)
    g = pl.num_programs(0)

    @pl.when(i == 0)
    def _init():
        counts_ref[...] = jnp.zeros_like(counts_ref)

    x = hs_ref[...]
    logits = jnp.dot(x, wt_ref[...], preferred_element_type=jnp.float32)
    lt = logits.T
    e, blk = lt.shape
    lt = lt + bias_ref[:, 0:1]
    probs_t = jax.nn.sigmoid(lt)
    gl = lt + bias_ref[:, 1:2]

    kb = jax.lax.bitcast_convert_type(gl, jnp.int32)
    key = kb ^ ((kb >> 31) & jnp.int32(0x7FFFFFFF))

    iota0 = jax.lax.broadcasted_iota(jnp.int32, (e, blk), 0)
    idx_rows = []
    p_rows = []
    for _ in range(_TOPK):
        m = jnp.max(key, axis=0, keepdims=True)
        eq = key == m
        idxk = jnp.min(jnp.where(eq, iota0, e), axis=0, keepdims=True)
        sel = iota0 == idxk
        pk = jnp.sum(jnp.where(sel, probs_t, 0.0), axis=0, keepdims=True)
        key = jnp.where(sel, jnp.int32(_IMIN), key)
        idx_rows.append(idxk)
        p_rows.append(pk)

    idx_t = jnp.concatenate(idx_rows, axis=0)
    p_t = jnp.concatenate(p_rows, axis=0)
    p_t = p_t / jnp.sum(p_t, axis=0, keepdims=True)
    idx_ref[...] = idx_t.T
    probs_ref[...] = p_t.T

    selected = (key == jnp.int32(_IMIN)).astype(jnp.float32)
    counts_ref[...] = counts_ref[...] + selected * mask_ref[...]

    @pl.when(i == g - 1)
    def _fin():
        c = jnp.sum(counts_ref[...], axis=1, keepdims=True)
        mx = jnp.max(c, axis=0, keepdims=True)
        avg = jnp.sum(c, axis=0, keepdims=True) / c.shape[0]
        vio_ref[...] = (mx - avg) / (avg + 1e-5)


@jax.jit
def kernel(hidden_states, mask, W, b, expert_biases):
    bb, tt, cc = hidden_states.shape
    ee = W.shape[0]
    n = bb * tt
    hs = hidden_states.reshape(n, cc)
    maskf = mask.reshape(1, n).astype(jnp.float32)
    wt = W.T
    bias2 = jnp.stack([b, expert_biases], axis=1)

    blk = 2048
    grid = n // blk
    idx, probs, vio = pl.pallas_call(
        _gate_kernel,
        grid=(grid,),
        in_specs=[
            pl.BlockSpec((blk, cc), lambda i: (i, 0)),
            pl.BlockSpec((cc, ee), lambda i: (0, 0)),
            pl.BlockSpec((ee, 2), lambda i: (0, 0)),
            pl.BlockSpec((1, blk), lambda i: (0, i)),
        ],
        out_specs=[
            pl.BlockSpec((blk, _TOPK), lambda i: (i, 0)),
            pl.BlockSpec((blk, _TOPK), lambda i: (i, 0)),
            pl.BlockSpec((1, 1), lambda i: (0, 0)),
        ],
        out_shape=[
            jax.ShapeDtypeStruct((n, _TOPK), jnp.int32),
            jax.ShapeDtypeStruct((n, _TOPK), jnp.float32),
            jax.ShapeDtypeStruct((1, 1), jnp.float32),
        ],
        scratch_shapes=[pltpu.VMEM((ee, blk), jnp.float32)],
    )(hs, wt, bias2, maskf)

    idx_flat = idx.reshape(-1)
    w_flat = jnp.broadcast_to(maskf.reshape(n, 1), (n, _TOPK)).reshape(-1)
    vio_sc = _sc_maxvio(idx_flat, w_flat)
    return idx, probs, vio_sc[0]


# speedup vs baseline: 1.5106x; 1.5106x over previous
import jax
import jax.numpy as jnp
from jax.experimental import pallas as pl
from jax.experimental.pallas import tpu as pltpu

_TOPK = 8
_IMIN = -2147483648


def _gate_kernel(hs_ref, wt_ref, bias_ref, mask_ref,
                 idx_ref, probs_ref, vio_ref, counts_ref):
    i = pl.program_id(0)
    g = pl.num_programs(0)

    @pl.when(i == 0)
    def _init():
        counts_ref[...] = jnp.zeros_like(counts_ref)

    x = hs_ref[...]
    lt = jax.lax.dot_general(wt_ref[...], x, (((0,), (1,)), ((), ())),
                             preferred_element_type=jnp.float32)
    e, blk = lt.shape
    lt = lt + bias_ref[:, 0:1]
    probs_t = jax.nn.sigmoid(lt)
    gl = lt + bias_ref[:, 1:2]

    kb = jax.lax.bitcast_convert_type(gl, jnp.int32)
    key = kb ^ ((kb >> 31) & jnp.int32(0x7FFFFFFF))

    iota0 = jax.lax.broadcasted_iota(jnp.int32, (e, blk), 0)
    idx_rows = []
    p_rows = []
    for _ in range(_TOPK):
        m = jnp.max(key, axis=0, keepdims=True)
        eq = key == m
        idxk = jnp.min(jnp.where(eq, iota0, e), axis=0, keepdims=True)
        sel = iota0 == idxk
        pk = jnp.sum(jnp.where(sel, probs_t, 0.0), axis=0, keepdims=True)
        key = jnp.where(sel, jnp.int32(_IMIN), key)
        idx_rows.append(idxk)
        p_rows.append(pk)

    idx_t = jnp.concatenate(idx_rows, axis=0)
    p_t = jnp.concatenate(p_rows, axis=0)
    p_t = p_t / jnp.sum(p_t, axis=0, keepdims=True)
    idx_ref[...] = idx_t.T
    probs_ref[...] = p_t.T

    selected = (key == jnp.int32(_IMIN)).astype(jnp.float32)
    counts_ref[...] = counts_ref[...] + selected * mask_ref[...]

    @pl.when(i == g - 1)
    def _fin():
        c = jnp.sum(counts_ref[...], axis=1, keepdims=True)
        mx = jnp.max(c, axis=0, keepdims=True)
        avg = jnp.sum(c, axis=0, keepdims=True) / c.shape[0]
        vio_ref[...] = (mx - avg) / (avg + 1e-5)


@jax.jit
def kernel(hidden_states, mask, W, b, expert_biases):
    bb, tt, cc = hidden_states.shape
    ee = W.shape[0]
    n = bb * tt
    hs = hidden_states.reshape(n, cc)
    maskf = mask.reshape(1, n).astype(jnp.float32)
    wt = W.T
    bias2 = jnp.stack([b, expert_biases], axis=1)

    blk = 2048
    grid = n // blk
    idx, probs, vio = pl.pallas_call(
        _gate_kernel,
        grid=(grid,),
        in_specs=[
            pl.BlockSpec((blk, cc), lambda i: (i, 0)),
            pl.BlockSpec((cc, ee), lambda i: (0, 0)),
            pl.BlockSpec((ee, 2), lambda i: (0, 0)),
            pl.BlockSpec((1, blk), lambda i: (0, i)),
        ],
        out_specs=[
            pl.BlockSpec((blk, _TOPK), lambda i: (i, 0)),
            pl.BlockSpec((blk, _TOPK), lambda i: (i, 0)),
            pl.BlockSpec((1, 1), lambda i: (0, 0)),
        ],
        out_shape=[
            jax.ShapeDtypeStruct((n, _TOPK), jnp.int32),
            jax.ShapeDtypeStruct((n, _TOPK), jnp.float32),
            jax.ShapeDtypeStruct((1, 1), jnp.float32),
        ],
        scratch_shapes=[pltpu.VMEM((ee, blk), jnp.float32)],
    )(hs, wt, bias2, maskf)
    return idx, probs, vio[0, 0]
